# baseline (speedup 1.0000x reference)
import functools

import jax
import jax.numpy as jnp
from jax import lax
from jax.experimental import pallas as pl
from jax.experimental.pallas import tpu as pltpu
from jax.experimental.pallas import tpu_sc as plsc

N = 10000
D = 128
NPAD = 10112
NCORES = 2
NSUB = 16
NW = NCORES * NSUB
E_REAL = 320000 + N
B = 64
CHUNKS = 162
PER_W = CHUNKS * B
EPAD = PER_W * NW
DROWS = 160
ROWS_PER_TILE = NPAD // NSUB
RB = 1264
NB = NPAD // RB


def _edge_pass_body(heads, xl_hbm, xr_hbm, src_hbm, dst_hbm, att_hbm,
                    feat_hbm, den_hbm,
                    att_v, srcv, dstv, xlv, xrv, zv,
                    den_tile, idxA, idxB, acc, den_sh, sem1, sem2):
  c = lax.axis_index("c")
  s = lax.axis_index("s")
  wid = c * NSUB + s

  zero16 = jnp.zeros((16,), jnp.float32)
  lane = lax.iota(jnp.int32, 16)

  for i in range(8):
    for k in range(D // 16):
      zv[i, pl.ds(k * 16, 16)] = zero16
  base_r = s * ROWS_PER_TILE

  def zacc(j, _):
    pltpu.sync_copy(zv, acc.at[pl.ds(base_r + j * 8, 8)])
    return 0

  lax.fori_loop(0, ROWS_PER_TILE // 8, zacc, 0)

  def zden(r, _):
    for k in range(D // 16):
      den_tile[r, pl.ds(k * 16, 16)] = zero16
    return 0

  lax.fori_loop(0, DROWS, zden, 0)

  @pl.when(s == 0)
  def _():
    def zdsh(j, _):
      pltpu.sync_copy(zv, den_sh.at[pl.ds(j * 8, 8)])
      return 0
    lax.fori_loop(0, DROWS // 8, zdsh, 0)

  def ziota(j, _):
    idxA[pl.ds(j * 16, 16)] = lane + j * 16
    return 0

  lax.fori_loop(0, 8, ziota, 0)
  for j in range(2):
    idxB[pl.ds(j * 16, 16)] = lane + 128 + j * 16

  pltpu.sync_copy(att_hbm, att_v)
  plsc.subcore_barrier()

  def compute():
    xl_b, xr_b, dst_b = xlv, xrv, dstv

    def edge(e, _):
      xk = []
      t = []
      for k in range(8):
        xlk = xl_b[e, pl.ds(k * 16, 16)]
        a = xlk + xr_b[e, pl.ds(k * 16, 16)]
        l = jnp.maximum(a, 0.2 * a)
        xk.append(xlk)
        t.append(l * att_v[pl.ds(k * 16, 16)])
      if heads == 2:
        r0 = (t[0] + t[1]) + (t[2] + t[3])
        r1 = (t[4] + t[5]) + (t[6] + t[7])
        w0 = jnp.exp(jnp.full((16,), jnp.sum(r0), jnp.float32))
        w1 = jnp.exp(jnp.full((16,), jnp.sum(r1), jnp.float32))
        for k in range(4):
          xl_b[e, pl.ds(k * 16, 16)] = xk[k] * w0
        for k in range(4, 8):
          xl_b[e, pl.ds(k * 16, 16)] = xk[k] * w1
        denv = jnp.where(lane == 0, w0, jnp.where(lane == 1, w1, 0.0))
      else:
        r0 = ((t[0] + t[1]) + (t[2] + t[3])) + ((t[4] + t[5]) + (t[6] + t[7]))
        w0 = jnp.exp(jnp.full((16,), jnp.sum(r0), jnp.float32))
        for k in range(8):
          xl_b[e, pl.ds(k * 16, 16)] = xk[k] * w0
        denv = jnp.where(lane == 0, w0, 0.0)
      xr_b[e, pl.ds(0, 16)] = denv
      return 0

    lax.fori_loop(0, B, edge, 0)
    pltpu.sync_copy(xl_b, acc.at[dst_b], add=True)

    def denupd(j, _):
      ids = lane + j * 16
      dvec = dst_b[pl.ds(j * 16, 16)]
      f0 = dvec * 2
      w0v = plsc.load_gather(xr_b, [ids, jnp.zeros((16,), jnp.int32)])
      plsc.addupdate_scatter(
          den_tile, [lax.shift_right_logical(f0, 7), f0 & 127], w0v)
      if heads == 2:
        f1 = f0 + 1
        w1v = plsc.load_gather(xr_b, [ids, jnp.ones((16,), jnp.int32)])
        plsc.addupdate_scatter(
            den_tile, [lax.shift_right_logical(f1, 7), f1 & 127], w1v)
      return 0

    lax.fori_loop(0, B // 16, denupd, 0)

  def chunk(g, _):
    base = wid * PER_W + g * B
    c1 = pltpu.async_copy(src_hbm.at[pl.ds(base, B)], srcv, sem1)
    c2 = pltpu.async_copy(dst_hbm.at[pl.ds(base, B)], dstv, sem2)
    c1.wait()
    c3 = pltpu.async_copy(xl_hbm.at[srcv], xlv, sem1)
    c2.wait()
    c4 = pltpu.async_copy(xr_hbm.at[dstv], xrv, sem2)
    c3.wait()
    c4.wait()
    compute()
    return 0

  lax.fori_loop(0, CHUNKS, chunk, 0)

  plsc.subcore_barrier()
  pltpu.sync_copy(den_tile.at[pl.ds(0, 128)], den_sh.at[idxA], add=True)
  pltpu.sync_copy(den_tile.at[pl.ds(128, 32)], den_sh.at[idxB], add=True)
  plsc.subcore_barrier()

  pltpu.sync_copy(acc.at[pl.ds(base_r, ROWS_PER_TILE)],
                  feat_hbm.at[c, pl.ds(base_r, ROWS_PER_TILE)])

  @pl.when(s < DROWS // 16)
  def _():
    pltpu.sync_copy(den_sh.at[pl.ds(s * 16, 16)],
                    den_hbm.at[c, pl.ds(s * 16, 16)])


def _make_edge_pass(heads):
  mesh = plsc.VectorSubcoreMesh(core_axis_name="c", subcore_axis_name="s")
  return pl.kernel(
      functools.partial(_edge_pass_body, heads),
      out_type=(
          jax.ShapeDtypeStruct((NCORES, NPAD, D), jnp.float32),
          jax.ShapeDtypeStruct((NCORES, DROWS, D), jnp.float32),
      ),
      mesh=mesh,
      compiler_params=pltpu.CompilerParams(needs_layout_passes=False),
      scratch_types=[
          pltpu.VMEM((D,), jnp.float32),
          pltpu.VMEM((B,), jnp.int32),
          pltpu.VMEM((B,), jnp.int32),
          pltpu.VMEM((B, D), jnp.float32),
          pltpu.VMEM((B, D), jnp.float32),
          pltpu.VMEM((8, D), jnp.float32),
          pltpu.VMEM((DROWS, D), jnp.float32),
          pltpu.VMEM((128,), jnp.int32),
          pltpu.VMEM((32,), jnp.int32),
          pltpu.VMEM_SHARED((NPAD, D), jnp.float32),
          pltpu.VMEM_SHARED((DROWS, D), jnp.float32),
          pltpu.SemaphoreType.DMA,
          pltpu.SemaphoreType.DMA,
      ],
      name=f"gat_edge_pass_h{heads}",
  )


_edge_pass_h2 = _make_edge_pass(2)
_edge_pass_h1 = _make_edge_pass(1)


def _proj_body(x_ref, wl_ref, wr_ref, xl_ref, xr_ref):
  xb = x_ref[...]
  xl_ref[...] = jnp.dot(xb, wl_ref[...], preferred_element_type=jnp.float32)
  xr_ref[...] = jnp.dot(xb, wr_ref[...], preferred_element_type=jnp.float32)


_proj = pl.pallas_call(
    _proj_body,
    grid=(NB,),
    in_specs=[
        pl.BlockSpec((RB, D), lambda i: (i, 0)),
        pl.BlockSpec((D, D), lambda i: (0, 0)),
        pl.BlockSpec((D, D), lambda i: (0, 0)),
    ],
    out_specs=[
        pl.BlockSpec((RB, D), lambda i: (i, 0)),
        pl.BlockSpec((RB, D), lambda i: (i, 0)),
    ],
    out_shape=[
        jax.ShapeDtypeStruct((NPAD, D), jnp.float32),
        jax.ShapeDtypeStruct((NPAD, D), jnp.float32),
    ],
)


def _mid_body(feat_ref, den_ref, b1_ref, g1_ref, be1_ref, wl2_ref, wr2_ref,
              xl2_ref, xr2_ref):
  raw = feat_ref[0] + feat_ref[1]
  den = den_ref[0] + den_ref[1]
  denb = jnp.concatenate(
      [jnp.broadcast_to(den[:, 0:1], (RB, 64)),
       jnp.broadcast_to(den[:, 1:2], (RB, 64))], axis=1)
  out = raw / (denb + 1e-16) + b1_ref[...]
  m = jnp.mean(out, axis=1, keepdims=True)
  v = jnp.mean((out - m) ** 2, axis=1, keepdims=True)
  h = (out - m) / jnp.sqrt(v + 1e-5) * g1_ref[...] + be1_ref[...]
  h = jnp.where(h > 0, h, jnp.exp(h) - 1.0)
  xl2_ref[...] = jnp.dot(h, wl2_ref[...], preferred_element_type=jnp.float32)
  xr2_ref[...] = jnp.dot(h, wr2_ref[...], preferred_element_type=jnp.float32)


_mid = pl.pallas_call(
    _mid_body,
    grid=(NB,),
    in_specs=[
        pl.BlockSpec((NCORES, RB, D), lambda i: (0, i, 0)),
        pl.BlockSpec((NCORES, RB, 2), lambda i: (0, i, 0)),
        pl.BlockSpec((1, D), lambda i: (0, 0)),
        pl.BlockSpec((1, D), lambda i: (0, 0)),
        pl.BlockSpec((1, D), lambda i: (0, 0)),
        pl.BlockSpec((D, D), lambda i: (0, 0)),
        pl.BlockSpec((D, D), lambda i: (0, 0)),
    ],
    out_specs=[
        pl.BlockSpec((RB, D), lambda i: (i, 0)),
        pl.BlockSpec((RB, D), lambda i: (i, 0)),
    ],
    out_shape=[
        jax.ShapeDtypeStruct((NPAD, D), jnp.float32),
        jax.ShapeDtypeStruct((NPAD, D), jnp.float32),
    ],
)


def _fin_body(feat_ref, den_ref, b2_ref, node_ref, graph_ref):
  i = pl.program_id(0)
  raw = feat_ref[0] + feat_ref[1]
  den = den_ref[0][:, 0:1] + den_ref[1][:, 0:1]
  node = raw / (den + 1e-16) + b2_ref[...]
  node_ref[...] = node
  rows = i * RB + lax.broadcasted_iota(jnp.int32, (RB, 1), 0)
  blksum = jnp.sum(jnp.where(rows < N, node, 0.0), axis=0, keepdims=True)
  tot = jnp.where(i == 0, 0.0, graph_ref[...]) + blksum
  graph_ref[...] = jnp.where(i == NB - 1, tot * (1.0 / N), tot)


_fin = pl.pallas_call(
    _fin_body,
    grid=(NB,),
    in_specs=[
        pl.BlockSpec((NCORES, RB, D), lambda i: (0, i, 0)),
        pl.BlockSpec((NCORES, RB, 2), lambda i: (0, i, 0)),
        pl.BlockSpec((1, D), lambda i: (0, 0)),
    ],
    out_specs=[
        pl.BlockSpec((RB, D), lambda i: (i, 0)),
        pl.BlockSpec((1, D), lambda i: (0, 0)),
    ],
    out_shape=[
        jax.ShapeDtypeStruct((NPAD, D), jnp.float32),
        jax.ShapeDtypeStruct((1, D), jnp.float32),
    ],
)


def _den_nodes(den_raw):
  return den_raw.reshape(NCORES, DROWS * D)[:, :2 * NPAD].reshape(
      NCORES, NPAD, 2)


def kernel(x, edge_index, Wl1, Wr1, att1, b1, g1, be1, Wl2, Wr2, att2, b2):
  f32 = jnp.float32
  x_pad = jnp.zeros((NPAD, D), f32).at[:N].set(x)
  loops = jnp.arange(N, dtype=edge_index.dtype)
  fill = jnp.full((EPAD - E_REAL,), N, dtype=edge_index.dtype)
  src = jnp.concatenate([edge_index[0], loops, fill])
  dst = jnp.concatenate([edge_index[1], loops, fill])

  xl1, xr1 = _proj(x_pad, Wl1, Wr1)
  feat1, den1 = _edge_pass_h2(xl1, xr1, src, dst, att1.reshape(D))
  xl2, xr2 = _mid(feat1, _den_nodes(den1), b1.reshape(1, D), g1.reshape(1, D),
                  be1.reshape(1, D), Wl2, Wr2)
  feat2, den2 = _edge_pass_h1(xl2, xr2, src, dst, att2.reshape(D))
  node_pad, graph = _fin(feat2, _den_nodes(den2), b2.reshape(1, D))
  return node_pad[:N], graph


# speedup vs baseline: 1.3071x; 1.3071x over previous
import functools

import jax
import jax.numpy as jnp
from jax import lax
from jax.experimental import pallas as pl
from jax.experimental.pallas import tpu as pltpu
from jax.experimental.pallas import tpu_sc as plsc

N = 10000
D = 128
NPAD = 10112
NCORES = 2
NSUB = 16
NW = NCORES * NSUB
E_REAL = 320000 + N
B = 64
CHUNKS = 162
PER_W = CHUNKS * B
EPAD = PER_W * NW
DROWS = 160
ROWS_PER_TILE = NPAD // NSUB
RB = 1264
NB = NPAD // RB


def _edge_pass_body(heads, xl_hbm, xr_hbm, src_hbm, dst_hbm, att_hbm,
                    feat_hbm, den_hbm,
                    att_v, srcv, dstv, xlv, xrv, zv,
                    den_tile, idxA, idxB, acc, den_sh, sem1, sem2):
  c = lax.axis_index("c")
  s = lax.axis_index("s")
  wid = c * NSUB + s

  zero16 = jnp.zeros((16,), jnp.float32)
  lane = lax.iota(jnp.int32, 16)

  for i in range(8):
    for k in range(D // 16):
      zv[i, pl.ds(k * 16, 16)] = zero16
  base_r = s * ROWS_PER_TILE

  def zacc(j, _):
    pltpu.sync_copy(zv, acc.at[pl.ds(base_r + j * 8, 8)])
    return 0

  lax.fori_loop(0, ROWS_PER_TILE // 8, zacc, 0)

  def zden(r, _):
    for k in range(D // 16):
      den_tile[r, pl.ds(k * 16, 16)] = zero16
    return 0

  lax.fori_loop(0, DROWS, zden, 0)

  @pl.when(s == 0)
  def _():
    def zdsh(j, _):
      pltpu.sync_copy(zv, den_sh.at[pl.ds(j * 8, 8)])
      return 0
    lax.fori_loop(0, DROWS // 8, zdsh, 0)

  def ziota(j, _):
    idxA[pl.ds(j * 16, 16)] = lane + j * 16
    return 0

  lax.fori_loop(0, 8, ziota, 0)
  for j in range(2):
    idxB[pl.ds(j * 16, 16)] = lane + 128 + j * 16

  pltpu.sync_copy(att_hbm, att_v)
  plsc.subcore_barrier()

  def compute():
    xl_b, xr_b, dst_b = xlv, xrv, dstv

    @plsc.parallel_loop(0, B, unroll=2)
    def edge(e):
      xk = []
      t = []
      for k in range(8):
        xlk = xl_b[e, pl.ds(k * 16, 16)]
        a = xlk + xr_b[e, pl.ds(k * 16, 16)]
        l = jnp.maximum(a, 0.2 * a)
        xk.append(xlk)
        t.append(l * att_v[pl.ds(k * 16, 16)])
      if heads == 2:
        r0 = (t[0] + t[1]) + (t[2] + t[3])
        r1 = (t[4] + t[5]) + (t[6] + t[7])
        w0 = jnp.exp(jnp.full((16,), jnp.sum(r0), jnp.float32))
        w1 = jnp.exp(jnp.full((16,), jnp.sum(r1), jnp.float32))
        for k in range(4):
          xl_b[e, pl.ds(k * 16, 16)] = xk[k] * w0
        for k in range(4, 8):
          xl_b[e, pl.ds(k * 16, 16)] = xk[k] * w1
        denv = jnp.where(lane == 0, w0, jnp.where(lane == 1, w1, 0.0))
      else:
        r0 = ((t[0] + t[1]) + (t[2] + t[3])) + ((t[4] + t[5]) + (t[6] + t[7]))
        w0 = jnp.exp(jnp.full((16,), jnp.sum(r0), jnp.float32))
        for k in range(8):
          xl_b[e, pl.ds(k * 16, 16)] = xk[k] * w0
        denv = jnp.where(lane == 0, w0, 0.0)
      xr_b[e, pl.ds(0, 16)] = denv

    pltpu.sync_copy(xl_b, acc.at[dst_b], add=True)

    def denupd(j, _):
      ids = lane + j * 16
      dvec = dst_b[pl.ds(j * 16, 16)]
      f0 = dvec * 2
      w0v = plsc.load_gather(xr_b, [ids, jnp.zeros((16,), jnp.int32)])
      plsc.addupdate_scatter(
          den_tile, [lax.shift_right_logical(f0, 7), f0 & 127], w0v)
      if heads == 2:
        f1 = f0 + 1
        w1v = plsc.load_gather(xr_b, [ids, jnp.ones((16,), jnp.int32)])
        plsc.addupdate_scatter(
            den_tile, [lax.shift_right_logical(f1, 7), f1 & 127], w1v)
      return 0

    lax.fori_loop(0, B // 16, denupd, 0)

  def chunk(g, _):
    base = wid * PER_W + g * B
    c1 = pltpu.async_copy(src_hbm.at[pl.ds(base, B)], srcv, sem1)
    c2 = pltpu.async_copy(dst_hbm.at[pl.ds(base, B)], dstv, sem2)
    c1.wait()
    c3 = pltpu.async_copy(xl_hbm.at[srcv], xlv, sem1)
    c2.wait()
    c4 = pltpu.async_copy(xr_hbm.at[dstv], xrv, sem2)
    c3.wait()
    c4.wait()
    compute()
    return 0

  lax.fori_loop(0, CHUNKS, chunk, 0)

  plsc.subcore_barrier()
  pltpu.sync_copy(den_tile.at[pl.ds(0, 128)], den_sh.at[idxA], add=True)
  pltpu.sync_copy(den_tile.at[pl.ds(128, 32)], den_sh.at[idxB], add=True)
  plsc.subcore_barrier()

  pltpu.sync_copy(acc.at[pl.ds(base_r, ROWS_PER_TILE)],
                  feat_hbm.at[c, pl.ds(base_r, ROWS_PER_TILE)])

  @pl.when(s < DROWS // 16)
  def _():
    pltpu.sync_copy(den_sh.at[pl.ds(s * 16, 16)],
                    den_hbm.at[c, pl.ds(s * 16, 16)])


def _make_edge_pass(heads):
  mesh = plsc.VectorSubcoreMesh(core_axis_name="c", subcore_axis_name="s")
  return pl.kernel(
      functools.partial(_edge_pass_body, heads),
      out_type=(
          jax.ShapeDtypeStruct((NCORES, NPAD, D), jnp.float32),
          jax.ShapeDtypeStruct((NCORES, DROWS, D), jnp.float32),
      ),
      mesh=mesh,
      compiler_params=pltpu.CompilerParams(needs_layout_passes=False),
      scratch_types=[
          pltpu.VMEM((D,), jnp.float32),
          pltpu.VMEM((B,), jnp.int32),
          pltpu.VMEM((B,), jnp.int32),
          pltpu.VMEM((B, D), jnp.float32),
          pltpu.VMEM((B, D), jnp.float32),
          pltpu.VMEM((8, D), jnp.float32),
          pltpu.VMEM((DROWS, D), jnp.float32),
          pltpu.VMEM((128,), jnp.int32),
          pltpu.VMEM((32,), jnp.int32),
          pltpu.VMEM_SHARED((NPAD, D), jnp.float32),
          pltpu.VMEM_SHARED((DROWS, D), jnp.float32),
          pltpu.SemaphoreType.DMA,
          pltpu.SemaphoreType.DMA,
      ],
      name=f"gat_edge_pass_h{heads}",
  )


_edge_pass_h2 = _make_edge_pass(2)
_edge_pass_h1 = _make_edge_pass(1)


def _proj_body(x_ref, wl_ref, wr_ref, xl_ref, xr_ref):
  xb = x_ref[...]
  xl_ref[...] = jnp.dot(xb, wl_ref[...], preferred_element_type=jnp.float32)
  xr_ref[...] = jnp.dot(xb, wr_ref[...], preferred_element_type=jnp.float32)


_proj = pl.pallas_call(
    _proj_body,
    grid=(NB,),
    in_specs=[
        pl.BlockSpec((RB, D), lambda i: (i, 0)),
        pl.BlockSpec((D, D), lambda i: (0, 0)),
        pl.BlockSpec((D, D), lambda i: (0, 0)),
    ],
    out_specs=[
        pl.BlockSpec((RB, D), lambda i: (i, 0)),
        pl.BlockSpec((RB, D), lambda i: (i, 0)),
    ],
    out_shape=[
        jax.ShapeDtypeStruct((NPAD, D), jnp.float32),
        jax.ShapeDtypeStruct((NPAD, D), jnp.float32),
    ],
)


def _mid_body(feat_ref, den_ref, b1_ref, g1_ref, be1_ref, wl2_ref, wr2_ref,
              xl2_ref, xr2_ref):
  raw = feat_ref[0] + feat_ref[1]
  den = den_ref[0] + den_ref[1]
  denb = jnp.concatenate(
      [jnp.broadcast_to(den[:, 0:1], (RB, 64)),
       jnp.broadcast_to(den[:, 1:2], (RB, 64))], axis=1)
  out = raw / (denb + 1e-16) + b1_ref[...]
  m = jnp.mean(out, axis=1, keepdims=True)
  v = jnp.mean((out - m) ** 2, axis=1, keepdims=True)
  h = (out - m) / jnp.sqrt(v + 1e-5) * g1_ref[...] + be1_ref[...]
  h = jnp.where(h > 0, h, jnp.exp(h) - 1.0)
  xl2_ref[...] = jnp.dot(h, wl2_ref[...], preferred_element_type=jnp.float32)
  xr2_ref[...] = jnp.dot(h, wr2_ref[...], preferred_element_type=jnp.float32)


_mid = pl.pallas_call(
    _mid_body,
    grid=(NB,),
    in_specs=[
        pl.BlockSpec((NCORES, RB, D), lambda i: (0, i, 0)),
        pl.BlockSpec((NCORES, RB, 2), lambda i: (0, i, 0)),
        pl.BlockSpec((1, D), lambda i: (0, 0)),
        pl.BlockSpec((1, D), lambda i: (0, 0)),
        pl.BlockSpec((1, D), lambda i: (0, 0)),
        pl.BlockSpec((D, D), lambda i: (0, 0)),
        pl.BlockSpec((D, D), lambda i: (0, 0)),
    ],
    out_specs=[
        pl.BlockSpec((RB, D), lambda i: (i, 0)),
        pl.BlockSpec((RB, D), lambda i: (i, 0)),
    ],
    out_shape=[
        jax.ShapeDtypeStruct((NPAD, D), jnp.float32),
        jax.ShapeDtypeStruct((NPAD, D), jnp.float32),
    ],
)


def _fin_body(feat_ref, den_ref, b2_ref, node_ref, graph_ref):
  i = pl.program_id(0)
  raw = feat_ref[0] + feat_ref[1]
  den = den_ref[0][:, 0:1] + den_ref[1][:, 0:1]
  node = raw / (den + 1e-16) + b2_ref[...]
  node_ref[...] = node
  rows = i * RB + lax.broadcasted_iota(jnp.int32, (RB, 1), 0)
  blksum = jnp.sum(jnp.where(rows < N, node, 0.0), axis=0, keepdims=True)
  tot = jnp.where(i == 0, 0.0, graph_ref[...]) + blksum
  graph_ref[...] = jnp.where(i == NB - 1, tot * (1.0 / N), tot)


_fin = pl.pallas_call(
    _fin_body,
    grid=(NB,),
    in_specs=[
        pl.BlockSpec((NCORES, RB, D), lambda i: (0, i, 0)),
        pl.BlockSpec((NCORES, RB, 2), lambda i: (0, i, 0)),
        pl.BlockSpec((1, D), lambda i: (0, 0)),
    ],
    out_specs=[
        pl.BlockSpec((RB, D), lambda i: (i, 0)),
        pl.BlockSpec((1, D), lambda i: (0, 0)),
    ],
    out_shape=[
        jax.ShapeDtypeStruct((NPAD, D), jnp.float32),
        jax.ShapeDtypeStruct((1, D), jnp.float32),
    ],
)


def _den_nodes(den_raw):
  return den_raw.reshape(NCORES, DROWS * D)[:, :2 * NPAD].reshape(
      NCORES, NPAD, 2)


def kernel(x, edge_index, Wl1, Wr1, att1, b1, g1, be1, Wl2, Wr2, att2, b2):
  f32 = jnp.float32
  x_pad = jnp.zeros((NPAD, D), f32).at[:N].set(x)
  loops = jnp.arange(N, dtype=edge_index.dtype)
  fill = jnp.full((EPAD - E_REAL,), N, dtype=edge_index.dtype)
  src = jnp.concatenate([edge_index[0], loops, fill])
  dst = jnp.concatenate([edge_index[1], loops, fill])

  xl1, xr1 = _proj(x_pad, Wl1, Wr1)
  feat1, den1 = _edge_pass_h2(xl1, xr1, src, dst, att1.reshape(D))
  xl2, xr2 = _mid(feat1, _den_nodes(den1), b1.reshape(1, D), g1.reshape(1, D),
                  be1.reshape(1, D), Wl2, Wr2)
  feat2, den2 = _edge_pass_h1(xl2, xr2, src, dst, att2.reshape(D))
  node_pad, graph = _fin(feat2, _den_nodes(den2), b2.reshape(1, D))
  return node_pad[:N], graph


# speedup vs baseline: 1.4154x; 1.0828x over previous
import functools

import jax
import jax.numpy as jnp
from jax import lax
from jax.experimental import pallas as pl
from jax.experimental.pallas import tpu as pltpu
from jax.experimental.pallas import tpu_sc as plsc

N = 10000
D = 128
NPAD = 10112
NCORES = 2
NSUB = 16
NW = NCORES * NSUB
E_REAL = 320000 + N
B = 96
CHUNKS = 108
PER_W = CHUNKS * B
EPAD = PER_W * NW
DROWS = 160
ROWS_PER_TILE = NPAD // NSUB
RB = 1264
NB = NPAD // RB


def _edge_pass_body(heads, xl_hbm, xr_hbm, src_hbm, dst_hbm, att_hbm,
                    feat_hbm, den_hbm,
                    att_v, srcv, dstv, xlv, xrv, zv,
                    den_tile, idxA, idxB, acc, den_sh, sem1, sem2):
  c = lax.axis_index("c")
  s = lax.axis_index("s")
  wid = c * NSUB + s

  zero16 = jnp.zeros((16,), jnp.float32)
  lane = lax.iota(jnp.int32, 16)

  for i in range(8):
    for k in range(D // 16):
      zv[i, pl.ds(k * 16, 16)] = zero16
  base_r = s * ROWS_PER_TILE

  def zacc(j, _):
    pltpu.sync_copy(zv, acc.at[pl.ds(base_r + j * 8, 8)])
    return 0

  lax.fori_loop(0, ROWS_PER_TILE // 8, zacc, 0)

  def zden(r, _):
    for k in range(D // 16):
      den_tile[r, pl.ds(k * 16, 16)] = zero16
    return 0

  lax.fori_loop(0, DROWS, zden, 0)

  @pl.when(s == 0)
  def _():
    def zdsh(j, _):
      pltpu.sync_copy(zv, den_sh.at[pl.ds(j * 8, 8)])
      return 0
    lax.fori_loop(0, DROWS // 8, zdsh, 0)

  def ziota(j, _):
    idxA[pl.ds(j * 16, 16)] = lane + j * 16
    return 0

  lax.fori_loop(0, 8, ziota, 0)
  for j in range(2):
    idxB[pl.ds(j * 16, 16)] = lane + 128 + j * 16

  pltpu.sync_copy(att_hbm, att_v)
  plsc.subcore_barrier()

  def compute():
    xl_b, xr_b, dst_b = xlv, xrv, dstv

    @plsc.parallel_loop(0, B, unroll=4)
    def edge(e):
      xk = []
      t = []
      for k in range(8):
        xlk = xl_b[e, pl.ds(k * 16, 16)]
        a = xlk + xr_b[e, pl.ds(k * 16, 16)]
        l = jnp.maximum(a, 0.2 * a)
        xk.append(xlk)
        t.append(l * att_v[pl.ds(k * 16, 16)])
      if heads == 2:
        r0 = (t[0] + t[1]) + (t[2] + t[3])
        r1 = (t[4] + t[5]) + (t[6] + t[7])
        w0 = jnp.exp(jnp.full((16,), jnp.sum(r0), jnp.float32))
        w1 = jnp.exp(jnp.full((16,), jnp.sum(r1), jnp.float32))
        for k in range(4):
          xl_b[e, pl.ds(k * 16, 16)] = xk[k] * w0
        for k in range(4, 8):
          xl_b[e, pl.ds(k * 16, 16)] = xk[k] * w1
        denv = jnp.where(lane == 0, w0, jnp.where(lane == 1, w1, 0.0))
      else:
        r0 = ((t[0] + t[1]) + (t[2] + t[3])) + ((t[4] + t[5]) + (t[6] + t[7]))
        w0 = jnp.exp(jnp.full((16,), jnp.sum(r0), jnp.float32))
        for k in range(8):
          xl_b[e, pl.ds(k * 16, 16)] = xk[k] * w0
        denv = jnp.where(lane == 0, w0, 0.0)
      xr_b[e, pl.ds(0, 16)] = denv

    pltpu.sync_copy(xl_b, acc.at[dst_b], add=True)

    def denupd(j, _):
      ids = lane + j * 16
      dvec = dst_b[pl.ds(j * 16, 16)]
      f0 = dvec * 2
      w0v = plsc.load_gather(xr_b, [ids, jnp.zeros((16,), jnp.int32)])
      plsc.addupdate_scatter(
          den_tile, [lax.shift_right_logical(f0, 7), f0 & 127], w0v)
      if heads == 2:
        f1 = f0 + 1
        w1v = plsc.load_gather(xr_b, [ids, jnp.ones((16,), jnp.int32)])
        plsc.addupdate_scatter(
            den_tile, [lax.shift_right_logical(f1, 7), f1 & 127], w1v)
      return 0

    lax.fori_loop(0, B // 16, denupd, 0)

  def chunk(g, _):
    base = wid * PER_W + g * B
    c1 = pltpu.async_copy(src_hbm.at[pl.ds(base, B)], srcv, sem1)
    c2 = pltpu.async_copy(dst_hbm.at[pl.ds(base, B)], dstv, sem2)
    c1.wait()
    c3 = pltpu.async_copy(xl_hbm.at[srcv], xlv, sem1)
    c2.wait()
    c4 = pltpu.async_copy(xr_hbm.at[dstv], xrv, sem2)
    c3.wait()
    c4.wait()
    compute()
    return 0

  lax.fori_loop(0, CHUNKS, chunk, 0)

  plsc.subcore_barrier()
  pltpu.sync_copy(den_tile.at[pl.ds(0, 128)], den_sh.at[idxA], add=True)
  pltpu.sync_copy(den_tile.at[pl.ds(128, 32)], den_sh.at[idxB], add=True)
  plsc.subcore_barrier()

  pltpu.sync_copy(acc.at[pl.ds(base_r, ROWS_PER_TILE)],
                  feat_hbm.at[c, pl.ds(base_r, ROWS_PER_TILE)])

  @pl.when(s < DROWS // 16)
  def _():
    pltpu.sync_copy(den_sh.at[pl.ds(s * 16, 16)],
                    den_hbm.at[c, pl.ds(s * 16, 16)])


def _make_edge_pass(heads):
  mesh = plsc.VectorSubcoreMesh(core_axis_name="c", subcore_axis_name="s")
  return pl.kernel(
      functools.partial(_edge_pass_body, heads),
      out_type=(
          jax.ShapeDtypeStruct((NCORES, NPAD, D), jnp.float32),
          jax.ShapeDtypeStruct((NCORES, DROWS, D), jnp.float32),
      ),
      mesh=mesh,
      compiler_params=pltpu.CompilerParams(needs_layout_passes=False),
      scratch_types=[
          pltpu.VMEM((D,), jnp.float32),
          pltpu.VMEM((B,), jnp.int32),
          pltpu.VMEM((B,), jnp.int32),
          pltpu.VMEM((B, D), jnp.float32),
          pltpu.VMEM((B, D), jnp.float32),
          pltpu.VMEM((8, D), jnp.float32),
          pltpu.VMEM((DROWS, D), jnp.float32),
          pltpu.VMEM((128,), jnp.int32),
          pltpu.VMEM((32,), jnp.int32),
          pltpu.VMEM_SHARED((NPAD, D), jnp.float32),
          pltpu.VMEM_SHARED((DROWS, D), jnp.float32),
          pltpu.SemaphoreType.DMA,
          pltpu.SemaphoreType.DMA,
      ],
      name=f"gat_edge_pass_h{heads}",
  )


_edge_pass_h2 = _make_edge_pass(2)
_edge_pass_h1 = _make_edge_pass(1)


def _proj_body(x_ref, wl_ref, wr_ref, xl_ref, xr_ref):
  xb = x_ref[...]
  xl_ref[...] = jnp.dot(xb, wl_ref[...], preferred_element_type=jnp.float32)
  xr_ref[...] = jnp.dot(xb, wr_ref[...], preferred_element_type=jnp.float32)


_proj = pl.pallas_call(
    _proj_body,
    grid=(NB,),
    in_specs=[
        pl.BlockSpec((RB, D), lambda i: (i, 0)),
        pl.BlockSpec((D, D), lambda i: (0, 0)),
        pl.BlockSpec((D, D), lambda i: (0, 0)),
    ],
    out_specs=[
        pl.BlockSpec((RB, D), lambda i: (i, 0)),
        pl.BlockSpec((RB, D), lambda i: (i, 0)),
    ],
    out_shape=[
        jax.ShapeDtypeStruct((NPAD, D), jnp.float32),
        jax.ShapeDtypeStruct((NPAD, D), jnp.float32),
    ],
)


def _mid_body(feat_ref, den_ref, b1_ref, g1_ref, be1_ref, wl2_ref, wr2_ref,
              xl2_ref, xr2_ref):
  raw = feat_ref[0] + feat_ref[1]
  den = den_ref[0] + den_ref[1]
  denb = jnp.concatenate(
      [jnp.broadcast_to(den[:, 0:1], (RB, 64)),
       jnp.broadcast_to(den[:, 1:2], (RB, 64))], axis=1)
  out = raw / (denb + 1e-16) + b1_ref[...]
  m = jnp.mean(out, axis=1, keepdims=True)
  v = jnp.mean((out - m) ** 2, axis=1, keepdims=True)
  h = (out - m) / jnp.sqrt(v + 1e-5) * g1_ref[...] + be1_ref[...]
  h = jnp.where(h > 0, h, jnp.exp(h) - 1.0)
  xl2_ref[...] = jnp.dot(h, wl2_ref[...], preferred_element_type=jnp.float32)
  xr2_ref[...] = jnp.dot(h, wr2_ref[...], preferred_element_type=jnp.float32)


_mid = pl.pallas_call(
    _mid_body,
    grid=(NB,),
    in_specs=[
        pl.BlockSpec((NCORES, RB, D), lambda i: (0, i, 0)),
        pl.BlockSpec((NCORES, RB, 2), lambda i: (0, i, 0)),
        pl.BlockSpec((1, D), lambda i: (0, 0)),
        pl.BlockSpec((1, D), lambda i: (0, 0)),
        pl.BlockSpec((1, D), lambda i: (0, 0)),
        pl.BlockSpec((D, D), lambda i: (0, 0)),
        pl.BlockSpec((D, D), lambda i: (0, 0)),
    ],
    out_specs=[
        pl.BlockSpec((RB, D), lambda i: (i, 0)),
        pl.BlockSpec((RB, D), lambda i: (i, 0)),
    ],
    out_shape=[
        jax.ShapeDtypeStruct((NPAD, D), jnp.float32),
        jax.ShapeDtypeStruct((NPAD, D), jnp.float32),
    ],
)


def _fin_body(feat_ref, den_ref, b2_ref, node_ref, graph_ref):
  i = pl.program_id(0)
  raw = feat_ref[0] + feat_ref[1]
  den = den_ref[0][:, 0:1] + den_ref[1][:, 0:1]
  node = raw / (den + 1e-16) + b2_ref[...]
  node_ref[...] = node
  rows = i * RB + lax.broadcasted_iota(jnp.int32, (RB, 1), 0)
  blksum = jnp.sum(jnp.where(rows < N, node, 0.0), axis=0, keepdims=True)
  tot = jnp.where(i == 0, 0.0, graph_ref[...]) + blksum
  graph_ref[...] = jnp.where(i == NB - 1, tot * (1.0 / N), tot)


_fin = pl.pallas_call(
    _fin_body,
    grid=(NB,),
    in_specs=[
        pl.BlockSpec((NCORES, RB, D), lambda i: (0, i, 0)),
        pl.BlockSpec((NCORES, RB, 2), lambda i: (0, i, 0)),
        pl.BlockSpec((1, D), lambda i: (0, 0)),
    ],
    out_specs=[
        pl.BlockSpec((RB, D), lambda i: (i, 0)),
        pl.BlockSpec((1, D), lambda i: (0, 0)),
    ],
    out_shape=[
        jax.ShapeDtypeStruct((NPAD, D), jnp.float32),
        jax.ShapeDtypeStruct((1, D), jnp.float32),
    ],
)


def _den_nodes(den_raw):
  return den_raw.reshape(NCORES, DROWS * D)[:, :2 * NPAD].reshape(
      NCORES, NPAD, 2)


def kernel(x, edge_index, Wl1, Wr1, att1, b1, g1, be1, Wl2, Wr2, att2, b2):
  f32 = jnp.float32
  x_pad = jnp.zeros((NPAD, D), f32).at[:N].set(x)
  loops = jnp.arange(N, dtype=edge_index.dtype)
  fill = jnp.full((EPAD - E_REAL,), N, dtype=edge_index.dtype)
  src = jnp.concatenate([edge_index[0], loops, fill])
  dst = jnp.concatenate([edge_index[1], loops, fill])

  xl1, xr1 = _proj(x_pad, Wl1, Wr1)
  feat1, den1 = _edge_pass_h2(xl1, xr1, src, dst, att1.reshape(D))
  xl2, xr2 = _mid(feat1, _den_nodes(den1), b1.reshape(1, D), g1.reshape(1, D),
                  be1.reshape(1, D), Wl2, Wr2)
  feat2, den2 = _edge_pass_h1(xl2, xr2, src, dst, att2.reshape(D))
  node_pad, graph = _fin(feat2, _den_nodes(den2), b2.reshape(1, D))
  return node_pad[:N], graph
